# Initial kernel scaffold; baseline (speedup 1.0000x reference)
import jax
import jax.numpy as jnp
from jax.experimental import pallas as pl


def kernel(x, table, W, b):
    raise NotImplementedError("write your pallas kernel here")


# speedup vs baseline: 1.5675x; 1.5675x over previous
import functools

import jax
import jax.numpy as jnp
from jax import lax
from jax.experimental import pallas as pl
from jax.experimental.pallas import tpu as pltpu
from jax.experimental.pallas import tpu_sc as plsc

VOCAB = 1_000_001
D = 32
B = 16384
S = 20

NC = 2
NS = 16
L = 16
NW = NC * NS

SAMPLES_PER_W = B // NW
CHUNK_SAMPLES = 64
CHUNK_ROWS = CHUNK_SAMPLES * S
NCHUNK = SAMPLES_PER_W // CHUNK_SAMPLES
GATHER_BLK = 128
NGATHER = CHUNK_ROWS // GATHER_BLK


def _sc_forward():
    mesh = plsc.VectorSubcoreMesh(core_axis_name="c", subcore_axis_name="s")

    @functools.partial(
        pl.kernel,
        mesh=mesh,
        compiler_params=pltpu.CompilerParams(
            needs_layout_passes=False, use_tc_tiling_on_sc=False),
        out_type=jax.ShapeDtypeStruct((B,), jnp.float32),
        scratch_types=[
            pltpu.VMEM((CHUNK_ROWS,), jnp.int32),
            pltpu.VMEM((CHUNK_ROWS,), jnp.int32),
            pltpu.VMEM((CHUNK_ROWS, D), jnp.float32),
            pltpu.VMEM((CHUNK_ROWS, D), jnp.float32),
            pltpu.VMEM((CHUNK_ROWS,), jnp.float32),
            pltpu.VMEM((SAMPLES_PER_W,), jnp.float32),
            pltpu.VMEM((D,), jnp.float32),
            pltpu.VMEM((L,), jnp.float32),
            pltpu.SemaphoreType.DMA,
            pltpu.SemaphoreType.DMA,
        ],
    )
    def k(x_ref, table_ref, w_ref, b_ref, out_ref,
          idx0, idx1, rows0, rows1, p_ref, acc_ref, w_v, b_v, sem0, sem1):
        wid = lax.axis_index("s") * NC + lax.axis_index("c")
        wbase = wid * (SAMPLES_PER_W * S)

        pltpu.sync_copy(w_ref, w_v)
        pltpu.sync_copy(b_ref, b_v)

        iota = lax.iota(jnp.int32, L)
        w_lo = w_v[pl.ds(0, L)]
        w_hi = w_v[pl.ds(L, L)]
        b_vec = b_v[pl.ds(0, L)]
        wsplat = [jnp.broadcast_to((w_lo if d < L else w_hi)[d % L], (L,))
                  for d in range(D)]
        bsplat = jnp.broadcast_to(b_vec[0], (L,))

        idx_bufs = (idx0, idx1)
        rows_bufs = (rows0, rows1)
        sems = (sem0, sem1)

        def stage_and_fire(c):
            nb = c % 2
            base = wbase + c * CHUNK_ROWS
            pltpu.sync_copy(x_ref.at[pl.ds(base, CHUNK_ROWS)], idx_bufs[nb])
            return [
                pltpu.async_copy(
                    table_ref.at[idx_bufs[nb].at[pl.ds(j * GATHER_BLK,
                                                       GATHER_BLK)]],
                    rows_bufs[nb].at[pl.ds(j * GATHER_BLK, GATHER_BLK)],
                    sems[nb])
                for j in range(NGATHER)
            ]

        def compute_chunk(c):
            nb = c % 2
            idx_b, rows_b = idx_bufs[nb], rows_bufs[nb]

            def t_body(t, carry):
                r0 = t * L
                iv = idx_b[pl.ds(r0, L)]
                row_i = iota + r0
                p = jnp.zeros((L,), jnp.float32)
                for d in range(D):
                    col = jnp.full((L,), d, jnp.int32)
                    p = p + plsc.load_gather(rows_b, [row_i, col]) * wsplat[d]
                p = jnp.where(iv != 0, p, 0.0)
                p_ref[pl.ds(r0, L)] = p
                return carry

            lax.fori_loop(0, CHUNK_ROWS // L, t_body, 0)

            def s_body(s, carry):
                lane_base = (iota + s * L) * S
                a = jnp.zeros((L,), jnp.float32)
                for j in range(S):
                    a = a + plsc.load_gather(p_ref, [lane_base + j])
                acc_ref[pl.ds(c * CHUNK_SAMPLES + s * L, L)] = a
                return carry

            lax.fori_loop(0, CHUNK_SAMPLES // L, s_body, 0)

        handles = stage_and_fire(0)
        for c in range(NCHUNK):
            nxt = stage_and_fire(c + 1) if c + 1 < NCHUNK else None
            for h in handles:
                h.wait()
            compute_chunk(c)
            handles = nxt

        inv_s = jnp.float32(1.0 / S)

        def fin_body(s, carry):
            z = acc_ref[pl.ds(s * L, L)] * inv_s + bsplat
            acc_ref[pl.ds(s * L, L)] = 1.0 / (1.0 + jnp.exp(-z))
            return carry

        lax.fori_loop(0, SAMPLES_PER_W // L, fin_body, 0)
        pltpu.sync_copy(acc_ref,
                        out_ref.at[pl.ds(wid * SAMPLES_PER_W, SAMPLES_PER_W)])

    return k


_forward = _sc_forward()


def kernel(x, table, W, b):
    x_flat = x.reshape(-1).astype(jnp.int32)
    w_flat = W.reshape(-1).astype(jnp.float32)
    b_pad = jnp.broadcast_to(b.astype(jnp.float32), (L,))
    probs = _forward(x_flat, table, w_flat, b_pad)
    return probs.reshape(B, 1)
